# Initial kernel scaffold; baseline (speedup 1.0000x reference)
import jax
import jax.numpy as jnp
from jax.experimental import pallas as pl


def kernel(diagnoses_idx, procedures_idx, emb, W, b):
    raise NotImplementedError("write your pallas kernel here")


# speedup vs baseline: 62.4465x; 62.4465x over previous
import functools

import jax
import jax.numpy as jnp
from jax import lax
from jax.experimental import pallas as pl
from jax.experimental.pallas import tpu as pltpu
from jax.experimental.pallas import tpu_sc as plsc

B, V, L = 4096, 20, 50
VOCAB, D, NCLS = 100000, 64, 100
NIDX = V * L
NCHUNK = 8
CW = NIDX // NCHUNK
NC, NS = 2, 16
NW = NC * NS
P = B // NW
NBUF = 4
RU = 5
NP = 128


def _pool_body(emb_hbm, didx_hbm, pidx_hbm, out_hbm,
               dix_v, pix_v, r0, r1, r2, r3, obuf_v, sem):
    rbufs = (r0, r1, r2, r3)
    wid = lax.axis_index("s") * NC + lax.axis_index("c")
    base = wid * P

    def sample_body(s, carry):
        bidx = base + s
        pltpu.sync_copy(didx_hbm.at[bidx], dix_v)
        pltpu.sync_copy(pidx_hbm.at[bidx], pix_v)

        def start(k):
            t, j = divmod(k, NCHUNK)
            iv = dix_v if t == 0 else pix_v
            return pltpu.async_copy(emb_hbm.at[iv.at[j]], rbufs[k % NBUF], sem)

        cps = {}
        for k in range(NBUF - 1):
            cps[k] = start(k)

        for t in range(2):
            acc = (jnp.zeros((16,), jnp.float32),) * 4
            for j in range(NCHUNK):
                k = t * NCHUNK + j
                if k + NBUF - 1 < 2 * NCHUNK:
                    cps[k + NBUF - 1] = start(k + NBUF - 1)
                cps[k].wait()
                rbuf = rbufs[k % NBUF]

                def red(r, a, rbuf=rbuf):
                    a = list(a)
                    for u in range(RU):
                        rr = r * RU + u
                        for q in range(4):
                            a[q] = a[q] + rbuf[rr, pl.ds(q * 16, 16)]
                    return tuple(a)

                acc = lax.fori_loop(0, CW // RU, red, acc)
            for q in range(4):
                obuf_v[s, pl.ds(t * D + q * 16, 16)] = acc[q] * (1.0 / L)
        return carry

    lax.fori_loop(0, P, sample_body, 0)
    pltpu.sync_copy(obuf_v, out_hbm.at[pl.ds(base, P)])


def _matmul_body(x_ref, w_ref, b_ref, o_ref):
    o_ref[...] = lax.dot_general(
        x_ref[...], w_ref[...], (((1,), (1,)), ((), ())),
        preferred_element_type=jnp.float32,
    ) + b_ref[...]


@jax.jit
def kernel(diagnoses_idx, procedures_idx, emb, W, b):
    didx = diagnoses_idx.reshape(B, NCHUNK, CW).astype(jnp.int32)
    pidx = procedures_idx.reshape(B, NCHUNK, CW).astype(jnp.int32)

    pooled = pl.kernel(
        _pool_body,
        out_type=jax.ShapeDtypeStruct((B, 2 * D), jnp.float32),
        mesh=plsc.VectorSubcoreMesh(
            core_axis_name="c", subcore_axis_name="s",
            num_cores=NC, num_subcores=NS),
        scratch_types=[
            pltpu.VMEM((NCHUNK, CW), jnp.int32),
            pltpu.VMEM((NCHUNK, CW), jnp.int32),
            pltpu.VMEM((CW, D), jnp.float32),
            pltpu.VMEM((CW, D), jnp.float32),
            pltpu.VMEM((CW, D), jnp.float32),
            pltpu.VMEM((CW, D), jnp.float32),
            pltpu.VMEM((P, 2 * D), jnp.float32),
            pltpu.SemaphoreType.DMA,
        ],
        compiler_params=pltpu.CompilerParams(use_tc_tiling_on_sc=False),
    )(emb, didx, pidx)

    w_pad = jnp.zeros((NP, 2 * D), jnp.float32).at[:NCLS].set(W)
    b_pad = jnp.zeros((1, NP), jnp.float32).at[0, :NCLS].set(b)

    rows_per_blk = 256
    out = pl.pallas_call(
        _matmul_body,
        grid=(B // rows_per_blk,),
        in_specs=[
            pl.BlockSpec((rows_per_blk, 2 * D), lambda i: (i, 0)),
            pl.BlockSpec((NP, 2 * D), lambda i: (0, 0)),
            pl.BlockSpec((1, NP), lambda i: (0, 0)),
        ],
        out_specs=pl.BlockSpec((rows_per_blk, NP), lambda i: (i, 0)),
        out_shape=jax.ShapeDtypeStruct((B, NP), jnp.float32),
    )(pooled, w_pad, b_pad)
    return out[:, :NCLS]


# speedup vs baseline: 79.5437x; 1.2738x over previous
import functools

import jax
import jax.numpy as jnp
import numpy as np
from jax import lax
from jax.experimental import pallas as pl
from jax.experimental.pallas import tpu as pltpu
from jax.experimental.pallas import tpu_sc as plsc

B, V, L = 4096, 20, 50
VOCAB, D, NCLS = 100000, 64, 100
NIDX = V * L
NCHUNK = 8
CW = NIDX // NCHUNK
DW = D // 2
NC, NS = 2, 16
NW = NC * NS
P = B // NW
RU = 5
NP = 128
_HI = -65536

_PERM = np.empty(2 * D, np.int32)
for _c in range(2 * D):
    _t, _r = divmod(_c, D)
    _g, _k = divmod(_r, 32)
    _PERM[_c] = _t * D + _g * 32 + (2 * _k if _k < 16 else 2 * (_k - 16) + 1)


def _pool_body(emb_hbm, idx_hbm, out_hbm, ix_v,
               r0, r1, r2, r3, r4, r5, r6, r7,
               r8, r9, r10, r11, r12, r13, r14, r15,
               obuf_v, semg, semi):
    rbufs = (r0, r1, r2, r3, r4, r5, r6, r7,
             r8, r9, r10, r11, r12, r13, r14, r15)
    wid = lax.axis_index("s") * NC + lax.axis_index("c")
    base = wid * P

    def idx_wait(p):
        pltpu.make_async_copy(idx_hbm.at[base], ix_v.at[p], semi).wait()

    def process(s, p):
        idx_wait(p)
        cps = []
        for t in range(2):
            for j in range(NCHUNK):
                k = t * NCHUNK + j
                cps.append(pltpu.async_copy(
                    emb_hbm.at[ix_v.at[p, t, j]], rbufs[k], semg))
        for t in range(2):
            acc = (jnp.zeros((16,), jnp.float32),) * 4
            for j in range(NCHUNK):
                k = t * NCHUNK + j
                cps[k].wait()
                rbuf = rbufs[k]

                def red(r, a, rbuf=rbuf):
                    a = list(a)
                    for u in range(RU):
                        rr = r * RU + u
                        for g in range(2):
                            w = rbuf[rr, pl.ds(g * 16, 16)]
                            a[2 * g] = a[2 * g] + plsc.bitcast(
                                w << 16, jnp.float32)
                            a[2 * g + 1] = a[2 * g + 1] + plsc.bitcast(
                                w & _HI, jnp.float32)
                    return tuple(a)

                acc = lax.fori_loop(0, CW // RU, red, acc)
            for q in range(4):
                obuf_v[s, pl.ds(t * D + q * 16, 16)] = acc[q] * (1.0 / L)

    def pair_body(i, carry):
        s0 = 2 * i
        pltpu.async_copy(idx_hbm.at[base + s0 + 1], ix_v.at[1], semi)
        process(s0, 0)
        nxt = base + lax.min(s0 + 2, P - 1)
        pltpu.async_copy(idx_hbm.at[nxt], ix_v.at[0], semi)
        process(s0 + 1, 1)
        return carry

    pltpu.async_copy(idx_hbm.at[base], ix_v.at[0], semi)
    lax.fori_loop(0, P // 2, pair_body, 0)
    idx_wait(0)
    pltpu.sync_copy(obuf_v, out_hbm.at[pl.ds(base, P)])


def _matmul_body(x_ref, w_ref, b_ref, o_ref):
    o_ref[...] = lax.dot_general(
        x_ref[...], w_ref[...], (((1,), (1,)), ((), ())),
        preferred_element_type=jnp.float32,
    ) + b_ref[...]


@jax.jit
def kernel(diagnoses_idx, procedures_idx, emb, W, b):
    idx = jnp.stack(
        [diagnoses_idx.reshape(B, NCHUNK, CW),
         procedures_idx.reshape(B, NCHUNK, CW)], axis=1).astype(jnp.int32)
    emb_pk = lax.bitcast_convert_type(
        emb.astype(jnp.bfloat16).reshape(VOCAB, DW, 2), jnp.int32)

    pooled = pl.kernel(
        _pool_body,
        out_type=jax.ShapeDtypeStruct((B, 2 * D), jnp.float32),
        mesh=plsc.VectorSubcoreMesh(
            core_axis_name="c", subcore_axis_name="s",
            num_cores=NC, num_subcores=NS),
        scratch_types=(
            [pltpu.VMEM((2, 2, NCHUNK, CW), jnp.int32)]
            + [pltpu.VMEM((CW, DW), jnp.int32) for _ in range(16)]
            + [pltpu.VMEM((P, 2 * D), jnp.float32),
               pltpu.SemaphoreType.DMA,
               pltpu.SemaphoreType.DMA]
        ),
        compiler_params=pltpu.CompilerParams(
            use_tc_tiling_on_sc=False, needs_layout_passes=False),
    )(emb_pk, idx)

    w_perm = W[:, _PERM]
    w_pad = jnp.zeros((NP, 2 * D), jnp.float32).at[:NCLS].set(w_perm)
    b_pad = jnp.zeros((1, NP), jnp.float32).at[0, :NCLS].set(b)

    rows_per_blk = 256
    out = pl.pallas_call(
        _matmul_body,
        grid=(B // rows_per_blk,),
        in_specs=[
            pl.BlockSpec((rows_per_blk, 2 * D), lambda i: (i, 0)),
            pl.BlockSpec((NP, 2 * D), lambda i: (0, 0)),
            pl.BlockSpec((1, NP), lambda i: (0, 0)),
        ],
        out_specs=pl.BlockSpec((rows_per_blk, NP), lambda i: (i, 0)),
        out_shape=jax.ShapeDtypeStruct((B, NP), jnp.float32),
    )(pooled, w_pad, b_pad)
    return out[:, :NCLS]


# speedup vs baseline: 98.0553x; 1.2327x over previous
import functools

import jax
import jax.numpy as jnp
import numpy as np
from jax import lax
from jax.experimental import pallas as pl
from jax.experimental.pallas import tpu as pltpu
from jax.experimental.pallas import tpu_sc as plsc

B, V, L = 4096, 20, 50
VOCAB, D, NCLS = 100000, 64, 100
NIDX = V * L
NCHUNK = 8
CW = NIDX // NCHUNK
DW = D // 2
NC, NS = 2, 16
NW = NC * NS
P = B // NW
RU = 5
NP = 128
_HI = -65536

_PERM = np.empty(2 * D, np.int32)
for _c in range(2 * D):
    _t, _r = divmod(_c, D)
    _g, _k = divmod(_r, 32)
    _PERM[_c] = _t * D + _g * 32 + (2 * _k if _k < 16 else 2 * (_k - 16) + 1)


def _pool_body(emb_hbm, didx_hbm, pidx_hbm, out_hbm, ix_v,
               r0, r1, r2, r3, r4, r5, r6, r7,
               r8, r9, r10, r11, r12, r13, r14, r15,
               obuf_v, semg, semi):
    rbufs = (r0, r1, r2, r3, r4, r5, r6, r7,
             r8, r9, r10, r11, r12, r13, r14, r15)
    wid = lax.axis_index("s") * NC + lax.axis_index("c")
    base = wid * P

    def idx_start(b, p):
        pltpu.async_copy(didx_hbm.at[b], ix_v.at[p, 0], semi)
        pltpu.async_copy(pidx_hbm.at[b], ix_v.at[p, 1], semi)

    def idx_wait(p):
        pltpu.make_async_copy(didx_hbm.at[base], ix_v.at[p, 0], semi).wait()
        pltpu.make_async_copy(pidx_hbm.at[base], ix_v.at[p, 1], semi).wait()

    def process(s, p):
        idx_wait(p)
        cps = []
        for t in range(2):
            for j in range(NCHUNK):
                k = t * NCHUNK + j
                cps.append(pltpu.async_copy(
                    emb_hbm.at[ix_v.at[p, t, j]], rbufs[k], semg))
        for t in range(2):
            acc = (jnp.zeros((16,), jnp.float32),) * 4
            for j in range(NCHUNK):
                k = t * NCHUNK + j
                cps[k].wait()
                rbuf = rbufs[k]

                def red(r, a, rbuf=rbuf):
                    a = list(a)
                    for u in range(RU):
                        rr = r * RU + u
                        for g in range(2):
                            w = plsc.bitcast(
                                rbuf[rr, pl.ds(g * 32, 32)], jnp.int32)
                            a[2 * g] = a[2 * g] + plsc.bitcast(
                                w << 16, jnp.float32)
                            a[2 * g + 1] = a[2 * g + 1] + plsc.bitcast(
                                w & _HI, jnp.float32)
                    return tuple(a)

                acc = lax.fori_loop(0, CW // RU, red, acc)
            for q in range(4):
                obuf_v[s, pl.ds(t * D + q * 16, 16)] = acc[q] * (1.0 / L)

    def pair_body(i, carry):
        s0 = 2 * i
        idx_start(base + s0 + 1, 1)
        process(s0, 0)
        idx_start(base + lax.min(s0 + 2, P - 1), 0)
        process(s0 + 1, 1)
        return carry

    idx_start(base, 0)
    lax.fori_loop(0, P // 2, pair_body, 0)
    idx_wait(0)
    pltpu.sync_copy(obuf_v, out_hbm.at[pl.ds(base, P)])


def _matmul_body(x_ref, w_ref, b_ref, o_ref):
    o_ref[...] = lax.dot_general(
        x_ref[...], w_ref[...], (((1,), (1,)), ((), ())),
        preferred_element_type=jnp.float32,
    ) + b_ref[...]


@jax.jit
def kernel(diagnoses_idx, procedures_idx, emb, W, b):
    didx = diagnoses_idx.reshape(B, NCHUNK, CW).astype(jnp.int32)
    pidx = procedures_idx.reshape(B, NCHUNK, CW).astype(jnp.int32)
    emb_bf = emb.astype(jnp.bfloat16)

    pooled = pl.kernel(
        _pool_body,
        out_type=jax.ShapeDtypeStruct((B, 2 * D), jnp.float32),
        mesh=plsc.VectorSubcoreMesh(
            core_axis_name="c", subcore_axis_name="s",
            num_cores=NC, num_subcores=NS),
        scratch_types=(
            [pltpu.VMEM((2, 2, NCHUNK, CW), jnp.int32)]
            + [pltpu.VMEM((CW, D), jnp.bfloat16) for _ in range(16)]
            + [pltpu.VMEM((P, 2 * D), jnp.float32),
               pltpu.SemaphoreType.DMA,
               pltpu.SemaphoreType.DMA]
        ),
        compiler_params=pltpu.CompilerParams(
            use_tc_tiling_on_sc=False, needs_layout_passes=False),
    )(emb_bf, didx, pidx)

    w_perm = W[:, _PERM]
    w_pad = jnp.zeros((NP, 2 * D), jnp.float32).at[:NCLS].set(w_perm)
    b_pad = jnp.zeros((1, NP), jnp.float32).at[0, :NCLS].set(b)

    rows_per_blk = 256
    out = pl.pallas_call(
        _matmul_body,
        grid=(B // rows_per_blk,),
        in_specs=[
            pl.BlockSpec((rows_per_blk, 2 * D), lambda i: (i, 0)),
            pl.BlockSpec((NP, 2 * D), lambda i: (0, 0)),
            pl.BlockSpec((1, NP), lambda i: (0, 0)),
        ],
        out_specs=pl.BlockSpec((rows_per_blk, NP), lambda i: (i, 0)),
        out_shape=jax.ShapeDtypeStruct((B, NP), jnp.float32),
    )(pooled, w_pad, b_pad)
    return out[:, :NCLS]
